# baseline (device time: 14307 ns/iter reference)
import jax
import jax.numpy as jnp
from jax import lax
from jax.experimental import pallas as pl
from jax.experimental.pallas import tpu as pltpu

N_DEV = 4

_DIRS = ("R", "R", "L", "L", "R", "R", "L")


def kernel(x, Win0, Wout0, Win1, Wout1, Win2, Wout2):
    m_per, d = x.shape
    h_per = Win0.shape[1]
    M = N_DEV * m_per
    H = N_DEV * h_per

    def body(x_hbm, wi0_hbm, wo0_hbm, wi1_hbm, wo1_hbm, wi2_hbm, wo2_hbm,
             out_ref, xs, wis, wos, xg, wic, woc, res,
             copy_sems, out_sem, send_sems, recv_sems):
        my = lax.axis_index("i")
        left = lax.rem(my + N_DEV - 1, N_DEV)
        right = lax.rem(my + 1, N_DEV)

        def chunk(t, k):
            if t == 0:
                return xg.at[pl.ds(k * m_per, m_per), :]
            l, kind = (t - 1) // 2, (t - 1) % 2
            if kind == 0:
                return wic.at[l, :, pl.ds(k * h_per, h_per)]
            return woc.at[l, pl.ds(k * h_per, h_per), :]

        barrier = pltpu.get_barrier_semaphore()
        for nbr in (left, right):
            pl.semaphore_signal(barrier, inc=1, device_id=(nbr,),
                                device_id_type=pl.DeviceIdType.MESH)

        stage_pairs = (
            (x_hbm, xs), (wi0_hbm, wis.at[0]), (wo0_hbm, wos.at[0]),
            (wi1_hbm, wis.at[1]), (wo1_hbm, wos.at[1]),
            (wi2_hbm, wis.at[2]), (wo2_hbm, wos.at[2]))
        cps = []
        for i, (src, dst) in enumerate(stage_pairs):
            cp = pltpu.make_async_copy(src, dst, copy_sems.at[i])
            cp.start()
            cps.append(cp)

        pl.semaphore_wait(barrier, 2)

        rdmas = []

        def staged_bf16(t):
            if t == 0:
                return xs[:, :].astype(jnp.bfloat16)
            l, kind = (t - 1) // 2, (t - 1) % 2
            return (wis[l] if kind == 0 else wos[l]).astype(jnp.bfloat16)

        for t in range(7):
            cps[t].wait()
            chunk(t, 0)[...] = staged_bf16(t)
            for di, (nbr, dst_k) in enumerate(((right, 3), (left, 1))):
                idx = 2 * t + di
                r = pltpu.make_async_remote_copy(
                    src_ref=chunk(t, 0), dst_ref=chunk(t, dst_k),
                    send_sem=send_sems.at[idx], recv_sem=recv_sems.at[idx],
                    device_id=(nbr,), device_id_type=pl.DeviceIdType.MESH)
                r.start()
                rdmas.append(r)

        def wait_recv(t, k, idx):
            pltpu.make_async_remote_copy(
                src_ref=chunk(t, k), dst_ref=chunk(t, k),
                send_sem=send_sems.at[idx], recv_sem=recv_sems.at[idx],
                device_id=(left,),
                device_id_type=pl.DeviceIdType.MESH).wait_recv()

        for t in range(7):
            idx2 = 14 + t
            if _DIRS[t] == "R":
                wait_recv(t, 3, 2 * t)
                src_k, nbr = 3, right
            else:
                wait_recv(t, 1, 2 * t + 1)
                src_k, nbr = 1, left
            r = pltpu.make_async_remote_copy(
                src_ref=chunk(t, src_k), dst_ref=chunk(t, 2),
                send_sem=send_sems.at[idx2], recv_sem=recv_sems.at[idx2],
                device_id=(nbr,), device_id_type=pl.DeviceIdType.MESH)
            r.start()
            rdmas.append(r)

        def wait_rest(t):
            if _DIRS[t] == "R":
                wait_recv(t, 1, 2 * t + 1)
            else:
                wait_recv(t, 3, 2 * t)
            wait_recv(t, 2, 14 + t)

        wait_rest(0)
        xcur = xg[:, :]
        for l in range(3):
            wait_rest(1 + 2 * l)
            wait_rest(2 + 2 * l)
            h = jnp.dot(xcur, wic[l], preferred_element_type=jnp.float32)
            h = jnp.maximum(h, 0.0).astype(jnp.bfloat16)
            p = jnp.dot(h, woc[l], preferred_element_type=jnp.float32)
            if l < 2:
                xcur = p.astype(jnp.bfloat16)
            else:
                pb = p.astype(jnp.bfloat16)
                for k in range(N_DEV):
                    pos = lax.rem(my + k, N_DEV)
                    res[pl.ds(pos * m_per, m_per), :] = (
                        pb[k * m_per:(k + 1) * m_per, :])

        out_cp = pltpu.make_async_copy(res, out_ref, out_sem)
        out_cp.start()
        for r in rdmas:
            r.wait_send()
        out_cp.wait()

    bf = jnp.bfloat16
    hbm = pltpu.MemorySpace.HBM
    args = tuple(
        pltpu.with_memory_space_constraint(a, hbm)
        for a in (x, Win0, Wout0, Win1, Wout1, Win2, Wout2))
    return pl.pallas_call(
        body,
        out_shape=jax.ShapeDtypeStruct((M, d), bf),
        in_specs=[pl.BlockSpec(memory_space=pl.ANY)] * 7,
        out_specs=pl.BlockSpec(memory_space=pl.ANY),
        scratch_shapes=[
            pltpu.VMEM((m_per, d), jnp.float32),
            pltpu.VMEM((3, d, h_per), jnp.float32),
            pltpu.VMEM((3, h_per, d), jnp.float32),
            pltpu.VMEM((M, d), bf),
            pltpu.VMEM((3, d, H), bf),
            pltpu.VMEM((3, H, d), bf),
            pltpu.VMEM((M, d), bf),
            pltpu.SemaphoreType.DMA((7,)),
            pltpu.SemaphoreType.DMA,
            pltpu.SemaphoreType.DMA((21,)),
            pltpu.SemaphoreType.DMA((21,)),
        ],
        compiler_params=pltpu.CompilerParams(collective_id=0),
    )(*args)


# device time: 14090 ns/iter; 1.0154x vs baseline; 1.0154x over previous
import jax
import jax.numpy as jnp
from jax import lax
from jax.experimental import pallas as pl
from jax.experimental.pallas import tpu as pltpu

N_DEV = 4


def kernel(x, Win0, Wout0, Win1, Wout1, Win2, Wout2):
    m_per, d = x.shape
    h_per = Win0.shape[1]
    M = N_DEV * m_per
    H = N_DEV * h_per

    def body(x_hbm, wi0_hbm, wo0_hbm, wi1_hbm, wo1_hbm, wi2_hbm, wo2_hbm,
             out_ref, xs, wis, wos, xg, wic, woc,
             copy_sems, send_sems, recv_sems):
        my = lax.axis_index("i")
        left = lax.rem(my + N_DEV - 1, N_DEV)
        right = lax.rem(my + 1, N_DEV)

        half_m, half_h = m_per // 2, h_per // 2

        def chunk(t, k):
            if t == 0:
                return xg.at[pl.ds(k * m_per, m_per), :]
            l, kind = (t - 1) // 2, (t - 1) % 2
            if kind == 0:
                return wic.at[l, :, pl.ds(k * h_per, h_per)]
            return woc.at[l, pl.ds(k * h_per, h_per), :]

        def chunk_half(t, k, h):
            if t == 0:
                return xg.at[pl.ds(k * m_per + h * half_m, half_m), :]
            l, kind = (t - 1) // 2, (t - 1) % 2
            if kind == 0:
                return wic.at[l, :, pl.ds(k * h_per + h * half_h, half_h)]
            return woc.at[l, pl.ds(k * h_per + h * half_h, half_h), :]

        barrier = pltpu.get_barrier_semaphore()
        for nbr in (left, right):
            pl.semaphore_signal(barrier, inc=1, device_id=(nbr,),
                                device_id_type=pl.DeviceIdType.MESH)

        stage_pairs = (
            (x_hbm, xs), (wi0_hbm, wis.at[0]), (wo0_hbm, wos.at[0]),
            (wi1_hbm, wis.at[1]), (wo1_hbm, wos.at[1]),
            (wi2_hbm, wis.at[2]), (wo2_hbm, wos.at[2]))
        cps = []
        for i, (src, dst) in enumerate(stage_pairs):
            cp = pltpu.make_async_copy(src, dst, copy_sems.at[i])
            cp.start()
            cps.append(cp)

        pl.semaphore_wait(barrier, 2)

        rdmas = []

        def staged_bf16(t):
            if t == 0:
                return xs[:, :].astype(jnp.bfloat16)
            l, kind = (t - 1) // 2, (t - 1) % 2
            return (wis[l] if kind == 0 else wos[l]).astype(jnp.bfloat16)

        for t in range(7):
            cps[t].wait()
            chunk(t, 0)[...] = staged_bf16(t)
            for di, (nbr, dst_k) in enumerate(((right, 3), (left, 1))):
                idx = 2 * t + di
                r = pltpu.make_async_remote_copy(
                    src_ref=chunk(t, 0), dst_ref=chunk(t, dst_k),
                    send_sem=send_sems.at[idx], recv_sem=recv_sems.at[idx],
                    device_id=(nbr,), device_id_type=pl.DeviceIdType.MESH)
                r.start()
                rdmas.append(r)

        def wait_recv(ref, idx):
            pltpu.make_async_remote_copy(
                src_ref=ref, dst_ref=ref,
                send_sem=send_sems.at[idx], recv_sem=recv_sems.at[idx],
                device_id=(left,),
                device_id_type=pl.DeviceIdType.MESH).wait_recv()

        for t in range(7):
            for hf, (src_k, nbr, h1_idx, idx2) in enumerate(
                    ((3, right, 2 * t, 14 + t),
                     (1, left, 2 * t + 1, 21 + t))):
                wait_recv(chunk(t, src_k), h1_idx)
                r = pltpu.make_async_remote_copy(
                    src_ref=chunk_half(t, src_k, hf),
                    dst_ref=chunk_half(t, 2, hf),
                    send_sem=send_sems.at[idx2], recv_sem=recv_sems.at[idx2],
                    device_id=(nbr,), device_id_type=pl.DeviceIdType.MESH)
                r.start()
                rdmas.append(r)

        def wait_rest(t):
            wait_recv(chunk_half(t, 2, 0), 14 + t)
            wait_recv(chunk_half(t, 2, 1), 21 + t)

        wait_rest(0)
        xcur = xg[:, :]
        for l in range(3):
            wait_rest(1 + 2 * l)
            wait_rest(2 + 2 * l)
            h = jnp.dot(xcur, wic[l], preferred_element_type=jnp.float32)
            h = jnp.maximum(h, 0.0).astype(jnp.bfloat16)
            p = jnp.dot(h, woc[l], preferred_element_type=jnp.float32)
            if l < 2:
                xcur = p.astype(jnp.bfloat16)
            else:
                pb = p.astype(jnp.bfloat16)
                for k in range(N_DEV):
                    pos = lax.rem(my + k, N_DEV)
                    out_ref[pl.ds(pos * m_per, m_per), :] = (
                        pb[k * m_per:(k + 1) * m_per, :])

        for r in rdmas:
            r.wait_send()

    bf = jnp.bfloat16
    hbm = pltpu.MemorySpace.HBM
    args = tuple(
        pltpu.with_memory_space_constraint(a, hbm)
        for a in (x, Win0, Wout0, Win1, Wout1, Win2, Wout2))
    return pl.pallas_call(
        body,
        out_shape=jax.ShapeDtypeStruct((M, d), bf),
        in_specs=[pl.BlockSpec(memory_space=pl.ANY)] * 7,
        out_specs=pl.BlockSpec(memory_space=pltpu.VMEM),
        scratch_shapes=[
            pltpu.VMEM((m_per, d), jnp.float32),
            pltpu.VMEM((3, d, h_per), jnp.float32),
            pltpu.VMEM((3, h_per, d), jnp.float32),
            pltpu.VMEM((M, d), bf),
            pltpu.VMEM((3, d, H), bf),
            pltpu.VMEM((3, H, d), bf),
            pltpu.SemaphoreType.DMA((7,)),
            pltpu.SemaphoreType.DMA((28,)),
            pltpu.SemaphoreType.DMA((28,)),
        ],
        compiler_params=pltpu.CompilerParams(collective_id=0),
    )(*args)
